# baseline (speedup 1.0000x reference)
import jax
import jax.numpy as jnp
from jax import lax
from jax.experimental import pallas as pl
from jax.experimental.pallas import tpu as pltpu
from jax.experimental.pallas import tpu_sc as plsc

N = 1000000
D = 64
G = 512
BATCH = 4
NCELL = BATCH * G * G
OWNER_SHIFT = 15
CPB = 32768
NW = 32
LANES = 16
NROW = 62500
VPW = 1960
NPAD = NW * VPW * LANES
MLP_BLK = 12800
REG_ROWS = 79 * MLP_BLK
CHS = NPAD // 64


def _mlp_body(x_ref, w1_ref, b1_ref, w2_ref, b2_ref, o_ref):
    pid = pl.program_id(0)
    h = jnp.maximum(x_ref[...] @ w1_ref[...] + b1_ref[...], 0.0)
    r = jnp.maximum(h @ w2_ref[...] + b2_ref[...], 0.0)
    rows = pid * MLP_BLK + lax.broadcasted_iota(jnp.int32, (MLP_BLK, 1), 0)
    o_ref[...] = jnp.where(rows < N, r, 0.0)


def _mlp_call(x, W1, b1, W2, b2):
    return pl.pallas_call(
        _mlp_body,
        grid=(REG_ROWS // MLP_BLK,),
        in_specs=[
            pl.BlockSpec((MLP_BLK, D), lambda i: (i, 0)),
            pl.BlockSpec((D, 32), lambda i: (0, 0)),
            pl.BlockSpec((1, 32), lambda i: (0, 0)),
            pl.BlockSpec((32, 2), lambda i: (0, 0)),
            pl.BlockSpec((1, 2), lambda i: (0, 0)),
        ],
        out_specs=pl.BlockSpec((MLP_BLK, 2), lambda i: (i, 0)),
        out_shape=jax.ShapeDtypeStruct((REG_ROWS, 2), jnp.float32),
    )(x, W1, b1.reshape(1, 32), W2, b2.reshape(1, 2))


def _cells_body(b_ref, x_ref, y_ref, c_ref):
    i = pl.program_id(0)
    rows = i * VPW + lax.broadcasted_iota(jnp.int32, (VPW, LANES), 0)
    valid = rows < NROW
    cell = (b_ref[...] * G + x_ref[...]) * G + y_ref[...]
    c_ref[...] = jnp.where(valid, cell, NCELL)


def _cells_call(ib2, ix2, iy2):
    spec_in = pl.BlockSpec((VPW, LANES), lambda i: (i, 0))
    return pl.pallas_call(
        _cells_body,
        grid=(NW,),
        in_specs=[spec_in, spec_in, spec_in],
        out_specs=pl.BlockSpec((VPW, LANES), lambda i: (i, 0)),
        out_shape=jax.ShapeDtypeStruct((NW * VPW, LANES), jnp.int32),
    )(ib2, ix2, iy2)


_MESH = dict(core_axis_name="c", subcore_axis_name="s")


def _apply_body(cells_hbm, reg0_hbm, reg1_hbm, out0_hbm, out1_hbm,
                W, vals0, vals1, cbuf, sem):
    wid = lax.axis_index("s") * 2 + lax.axis_index("c")
    iota = lax.iota(jnp.int32, LANES)

    def initw(v, carry):
        W[pl.ds(v * 16, 16)] = jnp.full((LANES,), -1, jnp.int32)
        return carry

    lax.fori_loop(0, CPB // 16, initw, 0)

    def chunk_body(g, carry):
        pltpu.sync_copy(cells_hbm.at[pl.ds(g * CHS, CHS)], cbuf)
        base = g * CHS

        def vbody(v, c2):
            c = cbuf[pl.ds(v * 16, 16)]
            owner = lax.shift_right_logical(c, OWNER_SHIFT)
            mine = owner == wid

            @pl.when(jnp.any(mine))
            def _update():
                ii = base + v * 16 + iota
                local = jnp.bitwise_and(c, CPB - 1)
                w0 = plsc.load_gather(W, [local])

                def wcond(st):
                    _, m = st
                    return jnp.any(m)

                def wbody(st):
                    _, m = st
                    plsc.store_scatter(W, [local], ii, mask=m)
                    w2 = plsc.load_gather(W, [local])
                    return (w2, mine & (ii > w2))

                lax.while_loop(wcond, wbody, (w0, mine & (ii > w0)))

            return c2

        lax.fori_loop(0, CHS // 16, vbody, 0)
        return carry

    lax.fori_loop(0, NPAD // CHS, chunk_body, 0)

    def fixw(v, carry):
        w = W[pl.ds(v * 16, 16)]
        W[pl.ds(v * 16, 16)] = jnp.where(w < 0, jnp.full((LANES,), N, jnp.int32), w)
        return carry

    lax.fori_loop(0, CPB // 16, fixw, 0)

    def gbody(g, carry):
        pltpu.async_copy(reg0_hbm.at[W.at[pl.ds(g * 128, 128)]],
                         vals0.at[pl.ds(g * 128, 128)], sem)
        pltpu.async_copy(reg1_hbm.at[W.at[pl.ds(g * 128, 128)]],
                         vals1.at[pl.ds(g * 128, 128)], sem)

        @pl.when(g >= 8)
        def _drain_one():
            pltpu.make_async_copy(reg0_hbm.at[W.at[pl.ds(0, 128)]],
                                  vals0.at[pl.ds(0, 128)], sem).wait()
            pltpu.make_async_copy(reg1_hbm.at[W.at[pl.ds(0, 128)]],
                                  vals1.at[pl.ds(0, 128)], sem).wait()

        return carry

    lax.fori_loop(0, CPB // 128, gbody, 0)

    def gdrain(g, carry):
        pltpu.make_async_copy(reg0_hbm.at[W.at[pl.ds(0, 128)]],
                              vals0.at[pl.ds(0, 128)], sem).wait()
        pltpu.make_async_copy(reg1_hbm.at[W.at[pl.ds(0, 128)]],
                              vals1.at[pl.ds(0, 128)], sem).wait()
        return carry

    lax.fori_loop(0, 8, gdrain, 0)
    pltpu.sync_copy(vals0, out0_hbm.at[pl.ds(wid * CPB, CPB)])
    pltpu.sync_copy(vals1, out1_hbm.at[pl.ds(wid * CPB, CPB)])


def _apply_call(cells1, reg0, reg1):
    mesh = plsc.VectorSubcoreMesh(**_MESH)
    f = pl.kernel(
        _apply_body,
        out_type=(
            jax.ShapeDtypeStruct((NCELL,), jnp.float32),
            jax.ShapeDtypeStruct((NCELL,), jnp.float32),
        ),
        mesh=mesh,
        compiler_params=pltpu.CompilerParams(needs_layout_passes=False),
        scratch_types=[
            pltpu.VMEM((CPB,), jnp.int32),
            pltpu.VMEM((CPB,), jnp.float32),
            pltpu.VMEM((CPB,), jnp.float32),
            pltpu.VMEM((CHS,), jnp.int32),
            pltpu.SemaphoreType.DMA,
        ],
    )
    return f(cells1, reg0, reg1)


def kernel(x, inds_b, inds_x, inds_y, W1, b1, W2, b2):
    reg = _mlp_call(x, W1, b1, W2, b2)
    reg0 = reg[:, 0]
    reg1 = reg[:, 1]
    ib2 = inds_b.reshape(NROW, LANES)
    ix2 = inds_x.reshape(NROW, LANES)
    iy2 = inds_y.reshape(NROW, LANES)
    cells2d = _cells_call(ib2, ix2, iy2)
    out0, out1 = _apply_call(cells2d.reshape(-1), reg0, reg1)
    return jnp.stack([out0, out1], axis=-1).reshape(BATCH, G, G, 2)


# speedup vs baseline: 1.6277x; 1.6277x over previous
import jax
import jax.numpy as jnp
from jax import lax
from jax.experimental import pallas as pl
from jax.experimental.pallas import tpu as pltpu
from jax.experimental.pallas import tpu_sc as plsc

N = 1000000
D = 64
G = 512
BATCH = 4
NCELL = BATCH * G * G
OWNER_SHIFT = 15
CPB = 32768
NW = 32
LANES = 16
NROW = 62500
VPW = 1960
NPAD = NW * VPW * LANES
MLP_BLK = 12800
REG_ROWS = 79 * MLP_BLK
CHS = NPAD // 64


def _mlp_body(x_ref, w1_ref, b1_ref, w2_ref, b2_ref, o_ref):
    pid = pl.program_id(0)
    h = jnp.maximum(x_ref[...] @ w1_ref[...] + b1_ref[...], 0.0)
    r = jnp.maximum(h @ w2_ref[...] + b2_ref[...], 0.0)
    rows = pid * MLP_BLK + lax.broadcasted_iota(jnp.int32, (MLP_BLK, 1), 0)
    o_ref[...] = jnp.where(rows < N, r, 0.0)


def _mlp_call(x, W1, b1, W2, b2):
    return pl.pallas_call(
        _mlp_body,
        grid=(REG_ROWS // MLP_BLK,),
        in_specs=[
            pl.BlockSpec((MLP_BLK, D), lambda i: (i, 0)),
            pl.BlockSpec((D, 32), lambda i: (0, 0)),
            pl.BlockSpec((1, 32), lambda i: (0, 0)),
            pl.BlockSpec((32, 2), lambda i: (0, 0)),
            pl.BlockSpec((1, 2), lambda i: (0, 0)),
        ],
        out_specs=pl.BlockSpec((MLP_BLK, 2), lambda i: (i, 0)),
        out_shape=jax.ShapeDtypeStruct((REG_ROWS, 2), jnp.float32),
    )(x, W1, b1.reshape(1, 32), W2, b2.reshape(1, 2))


def _cells_body(b_ref, x_ref, y_ref, c_ref):
    i = pl.program_id(0)
    rows = i * VPW + lax.broadcasted_iota(jnp.int32, (VPW, LANES), 0)
    valid = rows < NROW
    cell = (b_ref[...] * G + x_ref[...]) * G + y_ref[...]
    c_ref[...] = jnp.where(valid, cell, NCELL)


def _cells_call(ib2, ix2, iy2):
    spec_in = pl.BlockSpec((VPW, LANES), lambda i: (i, 0))
    return pl.pallas_call(
        _cells_body,
        grid=(NW,),
        in_specs=[spec_in, spec_in, spec_in],
        out_specs=pl.BlockSpec((VPW, LANES), lambda i: (i, 0)),
        out_shape=jax.ShapeDtypeStruct((NW * VPW, LANES), jnp.int32),
    )(ib2, ix2, iy2)


_MESH = dict(core_axis_name="c", subcore_axis_name="s")


def _apply_body(cells_hbm, reg0_hbm, reg1_hbm, out0_hbm, out1_hbm,
                W, vals0, vals1, cbuf, sem):
    wid = lax.axis_index("s") * 2 + lax.axis_index("c")
    iota = lax.iota(jnp.int32, LANES)

    def initw(v, carry):
        W[pl.ds(v * 16, 16)] = jnp.full((LANES,), -1, jnp.int32)
        return carry

    lax.fori_loop(0, CPB // 16, initw, 0)

    def chunk_body(g, carry):
        pltpu.sync_copy(cells_hbm.at[pl.ds(g * CHS, CHS)], cbuf)
        base = g * CHS

        def vbody(v, c2):
            c = cbuf[pl.ds(v * 16, 16)]
            owner = lax.shift_right_logical(c, OWNER_SHIFT)
            mine = owner == wid
            ii = base + v * 16 + iota
            local = jnp.bitwise_and(c, CPB - 1)
            w0 = plsc.load_gather(W, [local])
            plsc.store_scatter(W, [local], ii, mask=mine & (ii > w0))
            return c2

        lax.fori_loop(0, CHS // 16, vbody, 0)
        return carry

    lax.fori_loop(0, NPAD // CHS, chunk_body, 0)

    def fixw(v, carry):
        w = W[pl.ds(v * 16, 16)]
        W[pl.ds(v * 16, 16)] = jnp.where(w < 0, jnp.full((LANES,), N, jnp.int32), w)
        return carry

    lax.fori_loop(0, CPB // 16, fixw, 0)

    def gbody(g, carry):
        pltpu.async_copy(reg0_hbm.at[W.at[pl.ds(g * 128, 128)]],
                         vals0.at[pl.ds(g * 128, 128)], sem)
        pltpu.async_copy(reg1_hbm.at[W.at[pl.ds(g * 128, 128)]],
                         vals1.at[pl.ds(g * 128, 128)], sem)

        @pl.when(g >= 8)
        def _drain_one():
            pltpu.make_async_copy(reg0_hbm.at[W.at[pl.ds(0, 128)]],
                                  vals0.at[pl.ds(0, 128)], sem).wait()
            pltpu.make_async_copy(reg1_hbm.at[W.at[pl.ds(0, 128)]],
                                  vals1.at[pl.ds(0, 128)], sem).wait()

        return carry

    lax.fori_loop(0, CPB // 128, gbody, 0)

    def gdrain(g, carry):
        pltpu.make_async_copy(reg0_hbm.at[W.at[pl.ds(0, 128)]],
                              vals0.at[pl.ds(0, 128)], sem).wait()
        pltpu.make_async_copy(reg1_hbm.at[W.at[pl.ds(0, 128)]],
                              vals1.at[pl.ds(0, 128)], sem).wait()
        return carry

    lax.fori_loop(0, 8, gdrain, 0)
    pltpu.sync_copy(vals0, out0_hbm.at[pl.ds(wid * CPB, CPB)])
    pltpu.sync_copy(vals1, out1_hbm.at[pl.ds(wid * CPB, CPB)])


def _apply_call(cells1, reg0, reg1):
    mesh = plsc.VectorSubcoreMesh(**_MESH)
    f = pl.kernel(
        _apply_body,
        out_type=(
            jax.ShapeDtypeStruct((NCELL,), jnp.float32),
            jax.ShapeDtypeStruct((NCELL,), jnp.float32),
        ),
        mesh=mesh,
        compiler_params=pltpu.CompilerParams(needs_layout_passes=False),
        scratch_types=[
            pltpu.VMEM((CPB,), jnp.int32),
            pltpu.VMEM((CPB,), jnp.float32),
            pltpu.VMEM((CPB,), jnp.float32),
            pltpu.VMEM((CHS,), jnp.int32),
            pltpu.SemaphoreType.DMA,
        ],
    )
    return f(cells1, reg0, reg1)


def kernel(x, inds_b, inds_x, inds_y, W1, b1, W2, b2):
    reg = _mlp_call(x, W1, b1, W2, b2)
    reg0 = reg[:, 0]
    reg1 = reg[:, 1]
    ib2 = inds_b.reshape(NROW, LANES)
    ix2 = inds_x.reshape(NROW, LANES)
    iy2 = inds_y.reshape(NROW, LANES)
    cells2d = _cells_call(ib2, ix2, iy2)
    out0, out1 = _apply_call(cells2d.reshape(-1), reg0, reg1)
    return jnp.stack([out0, out1], axis=-1).reshape(BATCH, G, G, 2)


# speedup vs baseline: 1.8989x; 1.1666x over previous
import jax
import jax.numpy as jnp
from jax import lax
from jax.experimental import pallas as pl
from jax.experimental.pallas import tpu as pltpu
from jax.experimental.pallas import tpu_sc as plsc

N = 1000000
D = 64
G = 512
BATCH = 4
NCELL = BATCH * G * G
OWNER_SHIFT = 15
CPB = 32768
NW = 32
LANES = 16
NROW = 62500
VPW = 1960
NPAD = NW * VPW * LANES
MLP_BLK = 12800
REG_ROWS = 79 * MLP_BLK
CHS = NPAD // 64


def _mlp_body(x_ref, w1_ref, b1_ref, w2_ref, b2_ref, o_ref):
    pid = pl.program_id(0)
    h = jnp.maximum(x_ref[...] @ w1_ref[...] + b1_ref[...], 0.0)
    r = jnp.maximum(h @ w2_ref[...] + b2_ref[...], 0.0)
    cols = pid * MLP_BLK + lax.broadcasted_iota(jnp.int32, (1, MLP_BLK), 1)
    o_ref[...] = jnp.where(cols < N, r.T, 0.0)


def _mlp_call(x, W1, b1, W2, b2):
    return pl.pallas_call(
        _mlp_body,
        grid=(REG_ROWS // MLP_BLK,),
        in_specs=[
            pl.BlockSpec((MLP_BLK, D), lambda i: (i, 0)),
            pl.BlockSpec((D, 32), lambda i: (0, 0)),
            pl.BlockSpec((1, 32), lambda i: (0, 0)),
            pl.BlockSpec((32, 2), lambda i: (0, 0)),
            pl.BlockSpec((1, 2), lambda i: (0, 0)),
        ],
        out_specs=pl.BlockSpec((2, MLP_BLK), lambda i: (0, i)),
        out_shape=jax.ShapeDtypeStruct((2, REG_ROWS), jnp.float32),
    )(x, W1, b1.reshape(1, 32), W2, b2.reshape(1, 2))


def _cells_body(b_ref, x_ref, y_ref, c_ref):
    i = pl.program_id(0)
    rows = i * VPW + lax.broadcasted_iota(jnp.int32, (VPW, LANES), 0)
    valid = rows < NROW
    cell = (b_ref[...] * G + x_ref[...]) * G + y_ref[...]
    c_ref[...] = jnp.where(valid, cell, NCELL)


def _cells_call(ib2, ix2, iy2):
    spec_in = pl.BlockSpec((VPW, LANES), lambda i: (i, 0))
    return pl.pallas_call(
        _cells_body,
        grid=(NW,),
        in_specs=[spec_in, spec_in, spec_in],
        out_specs=pl.BlockSpec((VPW, LANES), lambda i: (i, 0)),
        out_shape=jax.ShapeDtypeStruct((NW * VPW, LANES), jnp.int32),
    )(ib2, ix2, iy2)


_MESH = dict(core_axis_name="c", subcore_axis_name="s")


def _apply_body(cells_hbm, reg0_hbm, reg1_hbm, out0_hbm, out1_hbm,
                W, vals0, vals1, cbuf, sem):
    wid = lax.axis_index("s") * 2 + lax.axis_index("c")
    iota = lax.iota(jnp.int32, LANES)

    def initw(v, carry):
        W[pl.ds(v * 16, 16)] = jnp.full((LANES,), -1, jnp.int32)
        return carry

    lax.fori_loop(0, CPB // 16, initw, 0)

    def chunk_body(g, carry):
        pltpu.sync_copy(cells_hbm.at[pl.ds(g * CHS, CHS)], cbuf)
        base = g * CHS

        def vbody(v, c2):
            for u in range(4):
                k = v * 4 + u
                c = cbuf[pl.ds(k * 16, 16)]
                owner = lax.shift_right_logical(c, OWNER_SHIFT)
                mine = owner == wid
                ii = base + k * 16 + iota
                local = jnp.bitwise_and(c, CPB - 1)
                w0 = plsc.load_gather(W, [local])
                plsc.store_scatter(W, [local], ii, mask=mine & (ii > w0))
            return c2

        lax.fori_loop(0, CHS // 64, vbody, 0)
        return carry

    lax.fori_loop(0, NPAD // CHS, chunk_body, 0)

    def fixw(v, carry):
        w = W[pl.ds(v * 16, 16)]
        W[pl.ds(v * 16, 16)] = jnp.where(w < 0, jnp.full((LANES,), N, jnp.int32), w)
        return carry

    lax.fori_loop(0, CPB // 16, fixw, 0)

    def gbody(g, carry):
        pltpu.async_copy(reg0_hbm.at[W.at[pl.ds(g * 128, 128)]],
                         vals0.at[pl.ds(g * 128, 128)], sem)
        pltpu.async_copy(reg1_hbm.at[W.at[pl.ds(g * 128, 128)]],
                         vals1.at[pl.ds(g * 128, 128)], sem)

        @pl.when(g >= 8)
        def _drain_one():
            pltpu.make_async_copy(reg0_hbm.at[W.at[pl.ds(0, 128)]],
                                  vals0.at[pl.ds(0, 128)], sem).wait()
            pltpu.make_async_copy(reg1_hbm.at[W.at[pl.ds(0, 128)]],
                                  vals1.at[pl.ds(0, 128)], sem).wait()

        return carry

    lax.fori_loop(0, CPB // 128, gbody, 0)

    def gdrain(g, carry):
        pltpu.make_async_copy(reg0_hbm.at[W.at[pl.ds(0, 128)]],
                              vals0.at[pl.ds(0, 128)], sem).wait()
        pltpu.make_async_copy(reg1_hbm.at[W.at[pl.ds(0, 128)]],
                              vals1.at[pl.ds(0, 128)], sem).wait()
        return carry

    lax.fori_loop(0, 8, gdrain, 0)
    pltpu.sync_copy(vals0, out0_hbm.at[pl.ds(wid * CPB, CPB)])
    pltpu.sync_copy(vals1, out1_hbm.at[pl.ds(wid * CPB, CPB)])


def _apply_call(cells1, reg0, reg1):
    mesh = plsc.VectorSubcoreMesh(**_MESH)
    f = pl.kernel(
        _apply_body,
        out_type=(
            jax.ShapeDtypeStruct((NCELL,), jnp.float32),
            jax.ShapeDtypeStruct((NCELL,), jnp.float32),
        ),
        mesh=mesh,
        compiler_params=pltpu.CompilerParams(needs_layout_passes=False),
        scratch_types=[
            pltpu.VMEM((CPB,), jnp.int32),
            pltpu.VMEM((CPB,), jnp.float32),
            pltpu.VMEM((CPB,), jnp.float32),
            pltpu.VMEM((CHS,), jnp.int32),
            pltpu.SemaphoreType.DMA,
        ],
    )
    return f(cells1, reg0, reg1)


def kernel(x, inds_b, inds_x, inds_y, W1, b1, W2, b2):
    reg = _mlp_call(x, W1, b1, W2, b2)
    reg0 = reg[0]
    reg1 = reg[1]
    ib2 = inds_b.reshape(NROW, LANES)
    ix2 = inds_x.reshape(NROW, LANES)
    iy2 = inds_y.reshape(NROW, LANES)
    cells2d = _cells_call(ib2, ix2, iy2)
    out0, out1 = _apply_call(cells2d.reshape(-1), reg0, reg1)
    return jnp.stack([out0, out1], axis=-1).reshape(BATCH, G, G, 2)
